# baseline (speedup 1.0000x reference)
import functools

import jax
import jax.numpy as jnp
from jax import lax
from jax.experimental import pallas as pl
from jax.experimental.pallas import tpu as pltpu
from jax.experimental.pallas import tpu_sc as plsc

_B = 4096
_L = 200
_DIM = 64
_NUM_EMB = 8192
_NTOT = _B * _L
_NC = 2
_NS = 16
_NW = _NC * _NS
_R = _DIM // 8
_Q = _NW // _R
_LQ = _L // _Q
_TBLW = 8 * _NUM_EMB


@functools.partial(
    pl.kernel,
    mesh=plsc.VectorSubcoreMesh(core_axis_name="c", subcore_axis_name="s"),
    out_type=jax.ShapeDtypeStruct((_L, 8, 32, 8, 128), jnp.float32),
    scratch_types=[
        pltpu.VMEM((_TBLW,), jnp.float32),
        pltpu.VMEM((_B,), jnp.int32),
        pltpu.VMEM((_B,), jnp.int32),
        pltpu.VMEM((16, 8, 128), jnp.float32),
        pltpu.VMEM((16, 8, 128), jnp.float32),
        pltpu.SemaphoreType.DMA,
        pltpu.SemaphoreType.DMA,
        pltpu.SemaphoreType.DMA,
        pltpu.SemaphoreType.DMA,
    ],
    compiler_params=pltpu.CompilerParams(
        use_tc_tiling_on_sc=True, needs_layout_passes=False
    ),
)
def _lookup(idx_hbm, table_hbm, out_hbm, tbl, ib0, ib1, ob0, ob1,
            si0, si1, so0, so1):
    wid = lax.axis_index("s") * _NC + lax.axis_index("c")
    r = wid % _R
    l0 = (wid // _R) * _LQ
    ib = (ib0, ib1)
    ob = (ob0, ob1)
    si = (si0, si1)
    so = (so0, so1)

    def fire_idx(li, p):
        pltpu.async_copy(idx_hbm.at[pl.ds((l0 + li) * _B, _B)], ib[p], si[p])

    def wait_idx(p):
        pltpu.make_async_copy(idx_hbm.at[pl.ds(0, _B)], ib[p], si[p]).wait()

    def build(ibuf, obuf, h):
        @plsc.parallel_loop(0, 16, unroll=4)
        def cbody(c):
            for gg in range(8):
                idxv = ibuf[pl.ds(h * 2048 + c * 128 + gg * 16, 16)]
                for d in range(8):
                    v = plsc.load_gather(tbl, [idxv + d * _NUM_EMB])
                    obuf[c, d, pl.ds(gg * 16, 16)] = v

    def fire_out(li, hb):
        pltpu.async_copy(
            ob[hb], out_hbm.at[l0 + li, r, pl.ds(16 * hb, 16)], so[hb]
        )

    def wait_out(hb):
        pltpu.make_async_copy(
            ob[hb], out_hbm.at[0, r, pl.ds(16 * hb, 16)], so[hb]
        ).wait()

    pltpu.sync_copy(table_hbm.at[pl.ds(r * _TBLW, _TBLW)], tbl)
    fire_idx(0, 0)
    fire_idx(1, 1)

    def do_l(li, p):
        wait_idx(p)

        @pl.when(li < _LQ - 1)
        def _():
            fire_idx(li + 1, 1 - p)

        for hb in range(2):
            @pl.when(li > 0)
            def _():
                wait_out(hb)
            build(ib[p], ob[hb], hb)
            fire_out(li, hb)

    def body(g2, carry):
        do_l(2 * g2, 0)
        do_l(2 * g2 + 1, 1)
        return carry

    lax.fori_loop(0, _LQ // 2, body, 0)
    wait_out(0)
    wait_out(1)


def kernel(idxes, pe):
    idx_t = idxes.astype(jnp.int32).T.reshape(_NTOT)
    pe_t = pe.T.reshape(_NUM_EMB * _DIM)
    out = _lookup(idx_t, pe_t)
    return jnp.transpose(out, (2, 4, 0, 1, 3)).reshape(_B, _L, _DIM)


# speedup vs baseline: 1.0562x; 1.0562x over previous
import functools

import jax
import jax.numpy as jnp
from jax import lax
from jax.experimental import pallas as pl
from jax.experimental.pallas import tpu as pltpu
from jax.experimental.pallas import tpu_sc as plsc

_B = 4096
_L = 200
_DIM = 64
_NUM_EMB = 8192
_NTOT = _B * _L
_NC = 2
_NS = 16
_NW = _NC * _NS
_R = _DIM // 8
_Q = _NW // _R
_LQ = _L // _Q
_TBLW = 8 * _NUM_EMB


@functools.partial(
    pl.kernel,
    mesh=plsc.VectorSubcoreMesh(core_axis_name="c", subcore_axis_name="s"),
    out_type=jax.ShapeDtypeStruct((_L, 8, 32, 8, 128), jnp.float32),
    scratch_types=[
        pltpu.VMEM((_TBLW,), jnp.float32),
        pltpu.VMEM((_B,), jnp.int32),
        pltpu.VMEM((_B,), jnp.int32),
        pltpu.VMEM((16, 8, 128), jnp.float32),
        pltpu.VMEM((16, 8, 128), jnp.float32),
        pltpu.SemaphoreType.DMA,
        pltpu.SemaphoreType.DMA,
        pltpu.SemaphoreType.DMA,
        pltpu.SemaphoreType.DMA,
    ],
    compiler_params=pltpu.CompilerParams(
        use_tc_tiling_on_sc=True, needs_layout_passes=False
    ),
)
def _lookup(idx_hbm, table_hbm, out_hbm, tbl, ib0, ib1, ob0, ob1,
            si0, si1, so0, so1):
    wid = lax.axis_index("s") * _NC + lax.axis_index("c")
    r = wid % _R
    l0 = (wid // _R) * _LQ
    ib = (ib0, ib1)
    ob = (ob0, ob1)
    si = (si0, si1)
    so = (so0, so1)

    def fire_idx(li, p):
        pltpu.async_copy(idx_hbm.at[pl.ds((l0 + li) * _B, _B)], ib[p], si[p])

    def wait_idx(p):
        pltpu.make_async_copy(idx_hbm.at[pl.ds(0, _B)], ib[p], si[p]).wait()

    def build(ibuf, obuf, h):
        @plsc.parallel_loop(0, 16, unroll=2)
        def cbody(c):
            for gg in range(8):
                idxv = ibuf[pl.ds(h * 2048 + c * 128 + gg * 16, 16)]
                for d in range(8):
                    v = plsc.load_gather(tbl, [idxv + d * _NUM_EMB])
                    obuf[c, d, pl.ds(gg * 16, 16)] = v

    def fire_out(li, hb):
        pltpu.async_copy(
            ob[hb], out_hbm.at[l0 + li, r, pl.ds(16 * hb, 16)], so[hb]
        )

    def wait_out(hb):
        pltpu.make_async_copy(
            ob[hb], out_hbm.at[0, r, pl.ds(16 * hb, 16)], so[hb]
        ).wait()

    pltpu.sync_copy(table_hbm.at[pl.ds(r * _TBLW, _TBLW)], tbl)
    fire_idx(0, 0)
    fire_idx(1, 1)

    def do_l(li, p):
        wait_idx(p)

        @pl.when(li < _LQ - 1)
        def _():
            fire_idx(li + 1, 1 - p)

        for hb in range(2):
            @pl.when(li > 0)
            def _():
                wait_out(hb)
            build(ib[p], ob[hb], hb)
            fire_out(li, hb)

    def body(g2, carry):
        do_l(2 * g2, 0)
        do_l(2 * g2 + 1, 1)
        return carry

    lax.fori_loop(0, _LQ // 2, body, 0)
    wait_out(0)
    wait_out(1)


def kernel(idxes, pe):
    idx_t = idxes.astype(jnp.int32).T.reshape(_NTOT)
    pe_t = pe.T.reshape(_NUM_EMB * _DIM)
    out = _lookup(idx_t, pe_t)
    return jnp.transpose(out, (2, 4, 0, 1, 3)).reshape(_B, _L, _DIM)


# speedup vs baseline: 1.1311x; 1.0710x over previous
import functools

import jax
import jax.numpy as jnp
from jax import lax
from jax.experimental import pallas as pl
from jax.experimental.pallas import tpu as pltpu
from jax.experimental.pallas import tpu_sc as plsc

_B = 4096
_L = 200
_DIM = 64
_NUM_EMB = 8192
_NTOT = _B * _L
_NC = 2
_NS = 16
_NW = _NC * _NS
_R = _DIM // 8
_Q = _NW // _R
_LQ = _L // _Q
_TBLW = 8 * _NUM_EMB


@functools.partial(
    pl.kernel,
    mesh=plsc.VectorSubcoreMesh(core_axis_name="c", subcore_axis_name="s"),
    out_type=jax.ShapeDtypeStruct((_L, 8, 32, 8, 128), jnp.float32),
    scratch_types=[
        pltpu.VMEM((_TBLW,), jnp.float32),
        pltpu.VMEM((_B,), jnp.int32),
        pltpu.VMEM((_B,), jnp.int32),
        pltpu.VMEM((16, 8, 128), jnp.float32),
        pltpu.VMEM((16, 8, 128), jnp.float32),
        pltpu.SemaphoreType.DMA,
        pltpu.SemaphoreType.DMA,
        pltpu.SemaphoreType.DMA,
        pltpu.SemaphoreType.DMA,
    ],
    compiler_params=pltpu.CompilerParams(
        use_tc_tiling_on_sc=True, needs_layout_passes=False
    ),
)
def _lookup(idx_hbm, table_hbm, out_hbm, tbl, ib0, ib1, ob0, ob1,
            si0, si1, so0, so1):
    wid = lax.axis_index("s") * _NC + lax.axis_index("c")
    r = wid % _R
    l0 = (wid // _R) * _LQ
    ib = (ib0, ib1)
    ob = (ob0, ob1)
    si = (si0, si1)
    so = (so0, so1)

    def fire_idx(li, p):
        pltpu.async_copy(idx_hbm.at[pl.ds((l0 + li) * _B, _B)], ib[p], si[p])

    def wait_idx(p):
        pltpu.make_async_copy(idx_hbm.at[pl.ds(0, _B)], ib[p], si[p]).wait()

    def build(ibuf, obuf, h):
        @plsc.parallel_loop(0, 16, unroll=1)
        def cbody(c):
            for gg in range(8):
                idxv = ibuf[pl.ds(h * 2048 + c * 128 + gg * 16, 16)]
                for d in range(8):
                    v = plsc.load_gather(tbl, [idxv + d * _NUM_EMB])
                    obuf[c, d, pl.ds(gg * 16, 16)] = v

    def fire_out(li, hb):
        pltpu.async_copy(
            ob[hb], out_hbm.at[l0 + li, r, pl.ds(16 * hb, 16)], so[hb]
        )

    def wait_out(hb):
        pltpu.make_async_copy(
            ob[hb], out_hbm.at[0, r, pl.ds(16 * hb, 16)], so[hb]
        ).wait()

    pltpu.sync_copy(table_hbm.at[pl.ds(r * _TBLW, _TBLW)], tbl)
    fire_idx(0, 0)
    fire_idx(1, 1)

    def do_l(li, p):
        wait_idx(p)

        @pl.when(li < _LQ - 1)
        def _():
            fire_idx(li + 1, 1 - p)

        for hb in range(2):
            @pl.when(li > 0)
            def _():
                wait_out(hb)
            build(ib[p], ob[hb], hb)
            fire_out(li, hb)

    def body(g2, carry):
        do_l(2 * g2, 0)
        do_l(2 * g2 + 1, 1)
        return carry

    lax.fori_loop(0, _LQ // 2, body, 0)
    wait_out(0)
    wait_out(1)


def kernel(idxes, pe):
    idx_t = idxes.astype(jnp.int32).T.reshape(_NTOT)
    pe_t = pe.T.reshape(_NUM_EMB * _DIM)
    out = _lookup(idx_t, pe_t)
    return jnp.transpose(out, (2, 4, 0, 1, 3)).reshape(_B, _L, _DIM)


# speedup vs baseline: 1.7961x; 1.5879x over previous
import functools

import jax
import jax.numpy as jnp
from jax import lax
from jax.experimental import pallas as pl
from jax.experimental.pallas import tpu as pltpu
from jax.experimental.pallas import tpu_sc as plsc

_B = 4096
_L = 200
_DIM = 64
_NUM_EMB = 8192
_NTOT = _B * _L
_NC = 2
_NS = 16
_NW = _NC * _NS
_R = _DIM // 8
_Q = _NW // _R
_LQ = _L // _Q
_TBLW = 8 * _NUM_EMB


@functools.partial(
    pl.kernel,
    mesh=plsc.VectorSubcoreMesh(core_axis_name="c", subcore_axis_name="s"),
    out_type=jax.ShapeDtypeStruct((_L, 8, 32, 8, 128), jnp.float32),
    scratch_types=[
        pltpu.VMEM((_TBLW,), jnp.float32),
        pltpu.VMEM((_B,), jnp.int32),
        pltpu.VMEM((_B,), jnp.int32),
        pltpu.VMEM((16, 8, 128), jnp.float32),
        pltpu.VMEM((16, 8, 128), jnp.float32),
        pltpu.SemaphoreType.DMA,
        pltpu.SemaphoreType.DMA,
        pltpu.SemaphoreType.DMA,
        pltpu.SemaphoreType.DMA,
    ],
    compiler_params=pltpu.CompilerParams(
        use_tc_tiling_on_sc=True, needs_layout_passes=False
    ),
)
def _lookup(idx_hbm, table_hbm, out_hbm, tbl, ib0, ib1, ob0, ob1,
            si0, si1, so0, so1):
    wid = lax.axis_index("s") * _NC + lax.axis_index("c")
    r = wid % _R
    l0 = (wid // _R) * _LQ
    ib = (ib0, ib1)
    ob = (ob0, ob1)
    si = (si0, si1)
    so = (so0, so1)

    def fire_idx(li, p):
        pltpu.async_copy(idx_hbm.at[pl.ds((l0 + li) * _B, _B)], ib[p], si[p])

    def wait_idx(p):
        pltpu.make_async_copy(idx_hbm.at[pl.ds(0, _B)], ib[p], si[p]).wait()

    def build(ibuf, obuf, h):
        @plsc.parallel_loop(0, 128, unroll=1)
        def gbody(i):
            idxv = ibuf[pl.ds(h * 2048 + i * 16, 16)]
            c = i // 8
            lane0 = (i % 8) * 16
            for d in range(8):
                v = plsc.load_gather(tbl, [idxv + d * _NUM_EMB])
                obuf[c, d, pl.ds(lane0, 16)] = v

    def fire_out(li, hb):
        pltpu.async_copy(
            ob[hb], out_hbm.at[l0 + li, r, pl.ds(16 * hb, 16)], so[hb]
        )

    def wait_out(hb):
        pltpu.make_async_copy(
            ob[hb], out_hbm.at[0, r, pl.ds(16 * hb, 16)], so[hb]
        ).wait()

    pltpu.sync_copy(table_hbm.at[pl.ds(r * _TBLW, _TBLW)], tbl)
    fire_idx(0, 0)
    fire_idx(1, 1)

    def do_l(li, p):
        wait_idx(p)

        @pl.when(li < _LQ - 1)
        def _():
            fire_idx(li + 1, 1 - p)

        for hb in range(2):
            @pl.when(li > 0)
            def _():
                wait_out(hb)
            build(ib[p], ob[hb], hb)
            fire_out(li, hb)

    def body(g2, carry):
        do_l(2 * g2, 0)
        do_l(2 * g2 + 1, 1)
        return carry

    lax.fori_loop(0, _LQ // 2, body, 0)
    wait_out(0)
    wait_out(1)


def kernel(idxes, pe):
    idx_t = idxes.astype(jnp.int32).T.reshape(_NTOT)
    pe_t = pe.T.reshape(_NUM_EMB * _DIM)
    out = _lookup(idx_t, pe_t)
    return jnp.transpose(out, (2, 4, 0, 1, 3)).reshape(_B, _L, _DIM)
